# Initial kernel scaffold; baseline (speedup 1.0000x reference)
import jax
import jax.numpy as jnp
from jax.experimental import pallas as pl


def kernel(sentence, table):
    raise NotImplementedError("write your pallas kernel here")


# speedup vs baseline: 1.4678x; 1.4678x over previous
import functools

import jax
import jax.numpy as jnp
from jax import lax
from jax.experimental import pallas as pl
from jax.experimental.pallas import tpu as pltpu
from jax.experimental.pallas import tpu_sc as plsc

VOCAB = 1000000
D = 32
BATCH = 4096
SEQ = 200

NC = 2
NS = 16
NW = NC * NS
IDX_W = 128
K = 10
CHUNK = K * IDX_W
ROWS_PER_W = (BATCH * SEQ) // NW
NCHUNK = ROWS_PER_W // CHUNK


def _make_gather():
    mesh = plsc.VectorSubcoreMesh(core_axis_name="c", subcore_axis_name="s")

    @functools.partial(
        pl.kernel,
        mesh=mesh,
        compiler_params=pltpu.CompilerParams(use_tc_tiling_on_sc=False),
        out_type=jax.ShapeDtypeStruct((NW * NCHUNK, K, IDX_W, D), jnp.float32),
        scratch_types=[
            pltpu.VMEM((K, IDX_W), jnp.int32),
            pltpu.VMEM((K, IDX_W, D), jnp.float32),
            pltpu.SemaphoreType.DMA,
        ],
    )
    def gather(idx_hbm, table_hbm, out_hbm, idx_v, rows_v, sem):
        wid = lax.axis_index("s") * NC + lax.axis_index("c")

        def chunk_body(g, carry):
            cid = wid * NCHUNK + g
            pltpu.sync_copy(idx_hbm.at[cid], idx_v)
            copies = [
                pltpu.async_copy(table_hbm.at[idx_v.at[j]], rows_v.at[j], sem)
                for j in range(K)
            ]
            for c in copies:
                c.wait()
            pltpu.sync_copy(rows_v, out_hbm.at[cid])
            return carry

        lax.fori_loop(0, NCHUNK, chunk_body, 0)

    return gather


_gather = _make_gather()


def kernel(sentence, table):
    idx = sentence.reshape(NW * NCHUNK, K, IDX_W).astype(jnp.int32)
    out = _gather(idx, table)
    return out.reshape(BATCH, SEQ, D)


# speedup vs baseline: 1.4937x; 1.0176x over previous
import functools

import jax
import jax.numpy as jnp
from jax import lax
from jax.experimental import pallas as pl
from jax.experimental.pallas import tpu as pltpu
from jax.experimental.pallas import tpu_sc as plsc

VOCAB = 1000000
D = 32
BATCH = 4096
SEQ = 200

NC = 2
NS = 16
NW = NC * NS
IDX_W = 128
K = 10
CHUNK = K * IDX_W
ROWS_PER_W = (BATCH * SEQ) // NW
NCHUNK = ROWS_PER_W // CHUNK
NSTREAM = ROWS_PER_W // IDX_W


def _make_gather():
    mesh = plsc.VectorSubcoreMesh(core_axis_name="c", subcore_axis_name="s")

    @functools.partial(
        pl.kernel,
        mesh=mesh,
        compiler_params=pltpu.CompilerParams(use_tc_tiling_on_sc=False),
        out_type=jax.ShapeDtypeStruct((NW * NCHUNK, K, IDX_W, D), jnp.float32),
        scratch_types=[
            pltpu.VMEM((NSTREAM, IDX_W), jnp.int32),
            pltpu.VMEM((2, K, IDX_W, D), jnp.float32),
            pltpu.SemaphoreType.DMA,
            pltpu.SemaphoreType.DMA,
            pltpu.SemaphoreType.DMA,
        ],
    )
    def gather(idx_hbm, table_hbm, out_hbm, idx_v, rows_v, gsem, osem0, osem1):
        wid = lax.axis_index("s") * NC + lax.axis_index("c")
        pltpu.sync_copy(idx_hbm.at[wid], idx_v)
        osems = (osem0, osem1)

        def outer(o, carry):
            for b in range(2):
                g = 2 * o + b
                cid = wid * NCHUNK + g
                buf = rows_v.at[b]
                @pl.when(g >= 2)
                def _():
                    pltpu.make_async_copy(buf, out_hbm.at[cid], osems[b]).wait()
                copies = [
                    pltpu.async_copy(
                        table_hbm.at[idx_v.at[g * K + j]], buf.at[j], gsem
                    )
                    for j in range(K)
                ]
                for c in copies:
                    c.wait()
                pltpu.async_copy(buf, out_hbm.at[cid], osems[b])
            return carry

        lax.fori_loop(0, NCHUNK // 2, outer, 0)
        for b in range(2):
            cid = wid * NCHUNK + (NCHUNK - 2 + b)
            pltpu.make_async_copy(rows_v.at[b], out_hbm.at[cid], osems[b]).wait()

    return gather


_gather = _make_gather()


def kernel(sentence, table):
    idx = sentence.reshape(NW, NSTREAM, IDX_W).astype(jnp.int32)
    out = _gather(idx, table)
    return out.reshape(BATCH, SEQ, D)


# speedup vs baseline: 1.4988x; 1.0034x over previous
import functools

import jax
import jax.numpy as jnp
from jax import lax
from jax.experimental import pallas as pl
from jax.experimental.pallas import tpu as pltpu
from jax.experimental.pallas import tpu_sc as plsc

VOCAB = 1000000
D = 32
BATCH = 4096
SEQ = 200

NC = 2
NS = 16
NW = NC * NS
BW = BATCH // NW
NB = 4
NCHUNK = BW // NB
HALVES = ((0, 128), (128, 72))
NBUF = 4


def _make_gather():
    mesh = plsc.VectorSubcoreMesh(core_axis_name="c", subcore_axis_name="s")

    @functools.partial(
        pl.kernel,
        mesh=mesh,
        compiler_params=pltpu.CompilerParams(use_tc_tiling_on_sc=False),
        out_type=jax.ShapeDtypeStruct((BATCH, SEQ, D), jnp.float32),
        scratch_types=[
            pltpu.VMEM((BW, SEQ), jnp.int32),
            pltpu.VMEM((NBUF, NB, SEQ, D), jnp.float32),
            [pltpu.SemaphoreType.DMA] * NBUF,
            [pltpu.SemaphoreType.DMA] * NBUF,
        ],
    )
    def gather(idx_hbm, table_hbm, out_hbm, idx_v, stage_v, gsems, osems):
        wid = lax.axis_index("s") * NC + lax.axis_index("c")
        pltpu.sync_copy(idx_hbm.at[wid], idx_v)

        def gather_copies(c, slot):
            out = []
            for i in range(NB):
                for off, width in HALVES:
                    out.append(
                        pltpu.make_async_copy(
                            table_hbm.at[
                                idx_v.at[c * NB + i, pl.ds(off, width)]
                            ],
                            stage_v.at[slot, i, pl.ds(off, width)],
                            gsems[slot],
                        )
                    )
            return out

        def out_copy(c, slot):
            return pltpu.make_async_copy(
                stage_v.at[slot],
                out_hbm.at[pl.ds(wid * BW + c * NB, NB)],
                osems[slot],
            )

        for c0 in range(2):
            for g in gather_copies(c0, c0):
                g.start()

        def outer(o, carry):
            for b in range(NBUF):
                c = NBUF * o + b
                for g in gather_copies(c, b):
                    g.wait()
                out_copy(c, b).start()
                nxt = (b + 2) % NBUF
                @pl.when(c >= 2)
                def _():
                    out_copy(c - 2, nxt).wait()
                @pl.when(c + 2 < NCHUNK)
                def _():
                    for g in gather_copies(c + 2, nxt):
                        g.start()
            return carry

        lax.fori_loop(0, NCHUNK // NBUF, outer, 0)
        for c0 in range(NCHUNK - 2, NCHUNK):
            out_copy(c0, c0 % NBUF).wait()

    return gather


_gather = _make_gather()


def kernel(sentence, table):
    idx = sentence.astype(jnp.int32).reshape(NW, BW, SEQ)
    return _gather(idx, table)


# speedup vs baseline: 2.0455x; 1.3647x over previous
import functools

import jax
import jax.numpy as jnp
from jax import lax
from jax.experimental import pallas as pl
from jax.experimental.pallas import tpu as pltpu
from jax.experimental.pallas import tpu_sc as plsc

VOCAB = 1000000
D = 32
BATCH = 4096
SEQ = 200

NC = 2
NS = 16
NW = NC * NS
BW = BATCH // NW
NB = 4
NCHUNK = BW // NB
HALVES = ((0, 128), (128, 72))
NBUF = 4


def _make_gather():
    mesh = plsc.VectorSubcoreMesh(core_axis_name="c", subcore_axis_name="s")

    @functools.partial(
        pl.kernel,
        mesh=mesh,
        compiler_params=pltpu.CompilerParams(use_tc_tiling_on_sc=False),
        out_type=jax.ShapeDtypeStruct((BATCH * SEQ, 4 * D), jnp.float32),
        scratch_types=[
            pltpu.VMEM((BW, SEQ), jnp.int32),
            pltpu.VMEM((NBUF, NB * SEQ, D), jnp.float32),
            [pltpu.SemaphoreType.DMA] * NBUF,
            [pltpu.SemaphoreType.DMA] * NBUF,
        ],
    )
    def gather(idx_hbm, table_hbm, out_hbm, idx_v, stage_v, gsems, osems):
        wid = lax.axis_index("s") * NC + lax.axis_index("c")
        pltpu.sync_copy(idx_hbm.at[wid], idx_v)

        def gather_copies(c, slot):
            out = []
            for i in range(NB):
                for off, width in HALVES:
                    out.append(
                        pltpu.make_async_copy(
                            table_hbm.at[
                                idx_v.at[c * NB + i, pl.ds(off, width)]
                            ],
                            stage_v.at[slot, pl.ds(i * SEQ + off, width)],
                            gsems[slot],
                        )
                    )
            return out

        def out_copy(c, slot):
            base = (wid * BW + c * NB) * SEQ
            return pltpu.make_async_copy(
                stage_v.at[slot],
                out_hbm.at[pl.ds(base, NB * SEQ), pl.ds(0, D)],
                osems[slot],
            )

        for c0 in range(2):
            for g in gather_copies(c0, c0):
                g.start()

        def outer(o, carry):
            for b in range(NBUF):
                c = NBUF * o + b
                for g in gather_copies(c, b):
                    g.wait()
                out_copy(c, b).start()
                nxt = (b + 2) % NBUF
                @pl.when(c >= 2)
                def _():
                    out_copy(c - 2, nxt).wait()
                @pl.when(c + 2 < NCHUNK)
                def _():
                    for g in gather_copies(c + 2, nxt):
                        g.start()
            return carry

        lax.fori_loop(0, NCHUNK // NBUF, outer, 0)
        for c0 in range(NCHUNK - 2, NCHUNK):
            out_copy(c0, c0 % NBUF).wait()

    return gather


_gather = _make_gather()


def kernel(sentence, table):
    idx = sentence.astype(jnp.int32).reshape(NW, BW, SEQ)
    out = _gather(idx, table)
    return out[:, :D].reshape(BATCH, SEQ, D)
